# Initial kernel scaffold; baseline (speedup 1.0000x reference)
import jax
import jax.numpy as jnp
from jax.experimental import pallas as pl


def kernel(input, l1_w0, l1_b0, l1_w1, l1_b1, l1_w2, l1_b2, l2_w0, l2_b0, l2_w1, l2_b1, l2_w2, l2_b2, l3_w0, l3_b0, l3_w1, l3_b1, l3_w2, l3_b2, d1_w, d1_b, d2_w, d2_b, d3_w, d3_b):
    raise NotImplementedError("write your pallas kernel here")


# speedup vs baseline: 13.9600x; 13.9600x over previous
import functools

import jax
import jax.numpy as jnp
from jax import lax
from jax.experimental import pallas as pl
from jax.experimental.pallas import tpu as pltpu
from jax.experimental.pallas import tpu_sc as plsc

BB = 32
NN = 2048
S1, G1 = 512, 32
S2, G2 = 128, 64
R1SQ = 0.2 * 0.2
R2SQ = 0.4 * 0.4
D1 = 16
D2 = 144


def _fps_body(n, m, xt_ref, out_ref):
    x = xt_ref[0, :, 0, :]
    y = xt_ref[1, :, 0, :]
    z = xt_ref[2, :, 0, :]
    iota = lax.broadcasted_iota(jnp.int32, (BB, n), 1)

    def step(i, carry):
        dist, far = carry
        oh = iota == far
        cx = jnp.sum(jnp.where(oh, x, 0.0), axis=1, keepdims=True)
        cy = jnp.sum(jnp.where(oh, y, 0.0), axis=1, keepdims=True)
        cz = jnp.sum(jnp.where(oh, z, 0.0), axis=1, keepdims=True)
        out_ref[:, pl.ds(i, 1), :] = jnp.concatenate([cx, cy, cz], axis=1)[:, None, :]
        d = (x - cx) ** 2 + (y - cy) ** 2 + (z - cz) ** 2
        dist = jnp.minimum(dist, d)
        mx = jnp.max(dist, axis=1, keepdims=True)
        far = jnp.min(jnp.where(dist == mx, iota, n), axis=1, keepdims=True)
        return dist, far.astype(jnp.int32)

    init = (jnp.full((BB, n), 1e10, jnp.float32), jnp.zeros((BB, 1), jnp.int32))
    lax.fori_loop(0, m, step, init)


def _fps(xt, m):
    n = xt.shape[3]
    return pl.pallas_call(
        functools.partial(_fps_body, n, m),
        out_shape=jax.ShapeDtypeStruct((BB, m, 3), jnp.float32),
    )(xt)


def _select_body(n, s, k, rsq, xt_ref, c_ref, idx_ref):
    b = pl.program_id(0)
    xs = xt_ref[:, 0, 0, :]
    c = c_ref[0]
    dx = c[:, 0:1] - xs[0:1]
    dy = c[:, 1:2] - xs[1:2]
    dz = c[:, 2:3] - xs[2:3]
    dist = dx * dx + dy * dy + dz * dz
    iota = lax.broadcasted_iota(jnp.int32, (s, n), 1)
    sent = 2 * n
    masked = jnp.where(dist <= rsq, iota, sent)
    first = jnp.min(masked, axis=1, keepdims=True)
    prev = jnp.full((s, 1), -1, jnp.int32)
    cols = []
    for _ in range(k):
        cur = jnp.min(jnp.where(masked > prev, masked, sent), axis=1, keepdims=True)
        cols.append(jnp.where(cur >= sent, first, cur))
        prev = jnp.where(cur >= sent, prev, cur)
    idx_ref[0] = jnp.concatenate(cols, axis=1) + b * n


def _select(xt, c, k, rsq):
    n = xt.shape[3]
    s = c.shape[1]
    return pl.pallas_call(
        functools.partial(_select_body, n, s, k, rsq),
        grid=(BB,),
        in_specs=[
            pl.BlockSpec((3, 1, 1, n), lambda b: (0, b, 0, 0)),
            pl.BlockSpec((1, s, 3), lambda b: (b, 0, 0)),
        ],
        out_specs=pl.BlockSpec((1, s, k), lambda b: (b, 0, 0)),
        out_shape=jax.ShapeDtypeStruct((BB, s, k), jnp.int32),
    )(xt, c)


def _sc_gather_rows(table, idx, chunk):
    info = plsc.get_sparse_core_info()
    nw = info.num_cores * info.num_subcores
    btot = idx.shape[0]
    d = table.shape[1]
    b_per_w = btot // nw
    mesh = plsc.VectorSubcoreMesh(core_axis_name="c", subcore_axis_name="s")

    @functools.partial(
        pl.kernel,
        mesh=mesh,
        out_type=jax.ShapeDtypeStruct((btot, d), jnp.float32),
        scratch_types=[
            pltpu.VMEM((b_per_w,), jnp.int32),
            pltpu.VMEM((chunk, d), jnp.float32),
            pltpu.SemaphoreType.DMA,
        ],
        compiler_params=pltpu.CompilerParams(use_tc_tiling_on_sc=False),
    )
    def gk(table_hbm, idx_hbm, out_hbm, idx_v, rows_v, sem):
        wid = lax.axis_index("s") * info.num_cores + lax.axis_index("c")
        base = pl.multiple_of(wid * b_per_w, 8)
        pltpu.sync_copy(idx_hbm.at[pl.ds(base, b_per_w)], idx_v)

        def step(j, carry):
            off = pl.multiple_of(j * chunk, 8)
            pltpu.async_copy(table_hbm.at[idx_v.at[pl.ds(off, chunk)]], rows_v, sem).wait()
            pltpu.sync_copy(rows_v, out_hbm.at[pl.ds(base + off, chunk)])
            return carry

        lax.fori_loop(0, b_per_w // chunk, step, 0)

    return gk(table, idx)


def _relu_mm(h, w, b):
    return jnp.maximum(jnp.dot(h, w, preferred_element_type=jnp.float32) + b, 0.0)


def _mlp1_body(g_ref, c_ref, w0, b0, w1, b1, w2, b2, t1_ref):
    g = g_ref[...]
    c = c_ref[0]
    g3 = g.reshape(S1, G1, D1)[:, :, 0:3]
    rel = (g3 - c[:, None, :]).reshape(S1 * G1, 3)
    h = _relu_mm(rel, w0[...], b0[...])
    h = _relu_mm(h, w1[...], b1[...])
    h = _relu_mm(h, w2[...], b2[...])
    pooled = jnp.max(h.reshape(S1, G1, 128), axis=1)
    pad = jnp.zeros((S1, D2 - 131), jnp.float32)
    t1_ref[0] = jnp.concatenate([c, pooled, pad], axis=1)


def _mlp1(g1, c1, w0, b0, w1, b1, w2, b2):
    wspec = lambda a: pl.BlockSpec(a.shape, lambda b: (0,) * a.ndim)
    return pl.pallas_call(
        _mlp1_body,
        grid=(BB,),
        in_specs=[
            pl.BlockSpec((S1 * G1, D1), lambda b: (b, 0)),
            pl.BlockSpec((1, S1, 3), lambda b: (b, 0, 0)),
            wspec(w0), wspec(b0), wspec(w1), wspec(b1), wspec(w2), wspec(b2),
        ],
        out_specs=pl.BlockSpec((1, S1, D2), lambda b: (b, 0, 0)),
        out_shape=jax.ShapeDtypeStruct((BB, S1, D2), jnp.float32),
    )(g1, c1, w0, b0, w1, b1, w2, b2)


def _mlp2_body(g_ref, c_ref, w0, b0, w1, b1, w2, b2, t2_ref):
    g = g_ref[...]
    c = c_ref[0]
    g3 = g.reshape(S2, G2, D2)
    rel = g3[:, :, 0:3] - c[:, None, :]
    inp = jnp.concatenate([rel, g3[:, :, 3:]], axis=2).reshape(S2 * G2, D2)
    h = _relu_mm(inp, w0[...], b0[...])
    h = _relu_mm(h, w1[...], b1[...])
    h = _relu_mm(h, w2[...], b2[...])
    pooled = jnp.max(h.reshape(S2, G2, 256), axis=1)
    t2_ref[0] = jnp.concatenate([c, pooled], axis=1)


def _mlp2(g2, c2, w0p, b0, w1, b1, w2, b2):
    wspec = lambda a: pl.BlockSpec(a.shape, lambda b: (0,) * a.ndim)
    return pl.pallas_call(
        _mlp2_body,
        grid=(BB,),
        in_specs=[
            pl.BlockSpec((S2 * G2, D2), lambda b: (b, 0)),
            pl.BlockSpec((1, S2, 3), lambda b: (b, 0, 0)),
            wspec(w0p), wspec(b0), wspec(w1), wspec(b1), wspec(w2), wspec(b2),
        ],
        out_specs=pl.BlockSpec((1, S2, 259), lambda b: (b, 0, 0)),
        out_shape=jax.ShapeDtypeStruct((BB, S2, 259), jnp.float32),
    )(g2, c2, w0p, b0, w1, b1, w2, b2)


def _head_body(t2_ref, w0, b0, w1, b1, w2, b2, d1w, d1b, d2w, d2b, d3w, d3b,
               out_ref):
    f = t2_ref[0]
    h = _relu_mm(f, w0[...], b0[...])
    h = _relu_mm(h, w1[...], b1[...])
    h = _relu_mm(h, w2[...], b2[...])
    v = jnp.max(h, axis=0, keepdims=True)
    v = _relu_mm(v, d1w[...], d1b[...])
    v = _relu_mm(v, d2w[...], d2b[...])
    logits = jnp.dot(v, d3w[...], preferred_element_type=jnp.float32) + d3b[...]
    m = jnp.max(logits, axis=1, keepdims=True)
    e = jnp.exp(logits - m)
    out_ref[0] = e / jnp.sum(e, axis=1, keepdims=True)


def _head(t2, w0, b0, w1, b1, w2, b2, d1w, d1b, d2w, d2b, d3w, d3b):
    ws = [w0, b0, w1, b1, w2, b2, d1w, d1b, d2w, d2b, d3w, d3b]
    wspec = lambda a: pl.BlockSpec(a.shape, lambda b: (0,) * a.ndim)
    return pl.pallas_call(
        _head_body,
        grid=(BB,),
        in_specs=[pl.BlockSpec((1, S2, 259), lambda b: (b, 0, 0))]
        + [wspec(a) for a in ws],
        out_specs=pl.BlockSpec((1, 1, 40), lambda b: (b, 0, 0)),
        out_shape=jax.ShapeDtypeStruct((BB, 1, 40), jnp.float32),
    )(t2, *ws)


def kernel(input, l1_w0, l1_b0, l1_w1, l1_b1, l1_w2, l1_b2,
           l2_w0, l2_b0, l2_w1, l2_b1, l2_w2, l2_b2,
           l3_w0, l3_b0, l3_w1, l3_b1, l3_w2, l3_b2,
           d1_w, d1_b, d2_w, d2_b, d3_w, d3_b):
    row = lambda b: b.reshape(1, -1)

    xt = jnp.transpose(input, (2, 0, 1))[:, :, None, :]
    c1 = _fps(xt, S1)
    idx1 = _select(xt, c1, G1, R1SQ)
    table1 = jnp.pad(input, ((0, 0), (0, 0), (0, D1 - 3))).reshape(BB * NN, D1)
    g1 = _sc_gather_rows(table1, idx1.reshape(-1), 2048)
    t1 = _mlp1(g1, c1, l1_w0, row(l1_b0), l1_w1, row(l1_b1),
               l1_w2, row(l1_b2))

    xt2 = jnp.transpose(c1, (2, 0, 1))[:, :, None, :]
    c2 = _fps(xt2, S2)
    idx2 = _select(xt2, c2, G2, R2SQ)
    g2 = _sc_gather_rows(t1.reshape(BB * S1, D2), idx2.reshape(-1), 512)
    w0p = jnp.pad(l2_w0, ((0, D2 - 131), (0, 0)))
    t2 = _mlp2(g2, c2, w0p, row(l2_b0), l2_w1, row(l2_b1),
               l2_w2, row(l2_b2))

    pred = _head(t2, l3_w0, row(l3_b0), l3_w1, row(l3_b1), l3_w2, row(l3_b2),
                 d1_w, row(d1_b), d2_w, row(d2_b), d3_w, row(d3_b))
    return pred.reshape(BB, 40)
